# baseline (speedup 1.0000x reference)
import functools

import jax
import jax.numpy as jnp
from jax import lax
from jax.experimental import pallas as pl
from jax.experimental.pallas import tpu as pltpu
from jax.experimental.pallas import tpu_sc as plsc

_CHUNK = 128


def _ceil_div(a, b):
    return (a + b - 1) // b


@functools.lru_cache(maxsize=None)
def _build(n_nodes, n_edges, hidden, n_ntypes, n_etypes):
    info = plsc.get_sparse_core_info()
    nc, ns = info.num_cores, info.num_subcores
    nw = nc * ns

    n_full, n_tail = n_nodes // _CHUNK, n_nodes % _CHUNK
    e_full, e_tail = n_edges // _CHUNK, n_edges % _CHUNK
    assert (n_full * _CHUNK) % 8 == 0 and (e_full * _CHUNK) % 8 == 0
    n_iters = _ceil_div(n_full, nw)
    e_iters = _ceil_div(e_full, nw)

    mesh = plsc.VectorSubcoreMesh(core_axis_name="c", subcore_axis_name="s")

    scratch = [
        pltpu.VMEM((_CHUNK,), jnp.int32),
        pltpu.VMEM((_CHUNK, hidden), jnp.float32),
        pltpu.SemaphoreType.DMA,
    ]
    if n_tail:
        scratch += [
            pltpu.VMEM((n_tail,), jnp.int32),
            pltpu.VMEM((n_tail, hidden), jnp.float32),
        ]
    if e_tail:
        scratch += [
            pltpu.VMEM((e_tail,), jnp.int32),
            pltpu.VMEM((e_tail, hidden), jnp.float32),
        ]

    @functools.partial(
        pl.kernel,
        mesh=mesh,
        out_type=jax.ShapeDtypeStruct((n_nodes + n_edges, hidden), jnp.float32),
        scratch_types=scratch,
    )
    def k(node_ids, edge_ids, ntab, etab, out, idx_v, rows_v, sem, *tails):
        wid = lax.axis_index("s") * nc + lax.axis_index("c")

        def do_chunk(ids_hbm, tab_hbm, out_base, c):
            off = c * _CHUNK
            pltpu.sync_copy(ids_hbm.at[pl.ds(off, _CHUNK)], idx_v)
            pltpu.async_copy(tab_hbm.at[idx_v], rows_v, sem).wait()
            pltpu.sync_copy(rows_v, out.at[pl.ds(out_base + off, _CHUNK)])

        def edge_body(i, _):
            c = i * nw + wid

            @pl.when(c < e_full)
            def _():
                do_chunk(edge_ids, etab, n_nodes, c)

            return 0

        lax.fori_loop(0, e_iters, edge_body, 0)

        def node_body(i, _):
            c = i * nw + wid

            @pl.when(c < n_full)
            def _():
                do_chunk(node_ids, ntab, 0, c)

            return 0

        lax.fori_loop(0, n_iters, node_body, 0)

        ti = 0
        if n_tail:
            idx_t, rows_t = tails[ti], tails[ti + 1]
            ti += 2

            @pl.when(wid == nw - 1)
            def _():
                off = n_full * _CHUNK
                pltpu.sync_copy(node_ids.at[pl.ds(off, n_tail)], idx_t)
                pltpu.async_copy(ntab.at[idx_t], rows_t, sem).wait()
                pltpu.sync_copy(rows_t, out.at[pl.ds(off, n_tail)])

        if e_tail:
            idx_t, rows_t = tails[ti], tails[ti + 1]

            @pl.when(wid == nw - 2)
            def _():
                off = e_full * _CHUNK
                pltpu.sync_copy(edge_ids.at[pl.ds(off, e_tail)], idx_t)
                pltpu.async_copy(etab.at[idx_t], rows_t, sem).wait()
                pltpu.sync_copy(rows_t, out.at[pl.ds(n_nodes + off, e_tail)])

    return k


def kernel(node_type_ids, edge_type_ids, node_type_table, edge_type_table):
    n_nodes = node_type_ids.shape[0]
    n_edges = edge_type_ids.shape[0]
    hidden = node_type_table.shape[1]
    k = _build(n_nodes, n_edges, hidden,
               node_type_table.shape[0], edge_type_table.shape[0])
    return k(node_type_ids.astype(jnp.int32), edge_type_ids.astype(jnp.int32),
             node_type_table, edge_type_table)
